# baseline (speedup 1.0000x reference)
import functools

import jax
import jax.numpy as jnp
from jax import lax
from jax.experimental import pallas as pl
from jax.experimental.pallas import tpu as pltpu
from jax.experimental.pallas import tpu_sc as plsc

DIM = 64
MARGIN = 1.0
BATCH = 16384
CHUNK = 128
LANES = 16


def _rsqrt(x):
    i = plsc.bitcast(x, jnp.int32)
    y = plsc.bitcast(jnp.int32(0x5F3759DF) - (i >> 1), jnp.float32)
    for _ in range(3):
        y = y * (1.5 - 0.5 * x * y * y)
    return y


def _make_sc_kernel(num_workers, bpw, nchunk):
    mesh = plsc.VectorSubcoreMesh(core_axis_name="c", subcore_axis_name="s")
    num_cores = mesh.num_cores

    @functools.partial(
        pl.kernel,
        mesh=mesh,
        compiler_params=pltpu.CompilerParams(needs_layout_passes=False,
                                             use_tc_tiling_on_sc=False),
        out_type=jax.ShapeDtypeStruct((num_workers, LANES), jnp.float32),
        scratch_types=[
            pltpu.VMEM((nchunk, CHUNK), jnp.int32),
            pltpu.VMEM((nchunk, CHUNK), jnp.int32),
            pltpu.VMEM((nchunk, CHUNK), jnp.int32),
            pltpu.VMEM((nchunk, CHUNK, DIM), jnp.float32),
            pltpu.VMEM((nchunk, CHUNK, DIM), jnp.float32),
            pltpu.VMEM((nchunk, CHUNK, DIM), jnp.float32),
            pltpu.VMEM((LANES,), jnp.float32),
            pltpu.SemaphoreType.DMA,
        ],
    )
    def trans_e_cost(lidx_hbm, ridx_hbm, qidx_hbm, ent_hbm, rel_hbm,
                     out_hbm, lv, rv, qv, lrows, rrows, qrows, outv, sem):
        wid = lax.axis_index("s") * num_cores + lax.axis_index("c")
        pltpu.sync_copy(lidx_hbm.at[wid], lv)
        pltpu.sync_copy(ridx_hbm.at[wid], rv)
        pltpu.sync_copy(qidx_hbm.at[wid], qv)
        copies = []
        for j in range(nchunk):
            copies.append(pltpu.async_copy(ent_hbm.at[lv.at[j]], lrows.at[j], sem))
            copies.append(pltpu.async_copy(ent_hbm.at[rv.at[j]], rrows.at[j], sem))
            copies.append(pltpu.async_copy(rel_hbm.at[qv.at[j]], qrows.at[j], sem))
        for c in copies:
            c.wait()

        iota = lax.iota(jnp.int32, LANES)
        zero = jnp.zeros((LANES,), jnp.float32)

        def group_body(g, acc):
            ji = jnp.full((LANES,), g >> 3, jnp.int32)
            ri = ((g & 7) << 4) + iota
            sl = sr = sq = dlr = dqr = zero
            for c in range(DIM):
                ci = jnp.full((LANES,), c, jnp.int32)
                lc = plsc.load_gather(lrows, [ji, ri, ci])
                rc = plsc.load_gather(rrows, [ji, ri, ci])
                qc = plsc.load_gather(qrows, [ji, ri, ci])
                sl = sl + lc * lc
                sr = sr + rc * rc
                sq = sq + qc * qc
                dlr = dlr + lc * rc
                dqr = dqr + qc * rc
            tiny = jnp.float32(1e-24)
            simi = (dlr * _rsqrt(jnp.maximum(sl * sr, tiny))
                    + dqr * _rsqrt(jnp.maximum(sq * sr, tiny)))
            similn = simi
            simirn = simi
            costl = jnp.maximum(similn - simi + MARGIN, 0.0)
            costr = jnp.maximum(simirn - simi + MARGIN, 0.0)
            return acc + costl + costr

        acc = lax.fori_loop(0, bpw // LANES, group_body, zero)
        total = jnp.sum(acc) * jnp.float32(1.0 / BATCH)
        outv[...] = jnp.where(iota == 0, total, 0.0)
        pltpu.sync_copy(outv, out_hbm.at[wid])

    return trans_e_cost


def kernel(leftEnIndices, rightEnIndices, relIndices, negLeftEnIndices,
           negRightEnIndices, entityEmbedding, relationEmbedding):
    del negLeftEnIndices, negRightEnIndices
    info = plsc.get_sparse_core_info()
    num_workers = info.num_cores * info.num_subcores
    bpw = BATCH // num_workers
    nchunk = bpw // CHUNK
    lidx = jnp.reshape(leftEnIndices.astype(jnp.int32), (num_workers, nchunk, CHUNK))
    ridx = jnp.reshape(rightEnIndices.astype(jnp.int32), (num_workers, nchunk, CHUNK))
    qidx = jnp.reshape(relIndices.astype(jnp.int32), (num_workers, nchunk, CHUNK))
    sc = _make_sc_kernel(num_workers, bpw, nchunk)
    partials = sc(lidx, ridx, qidx, entityEmbedding, relationEmbedding)
    return jnp.sum(partials)


# speedup vs baseline: 1.8674x; 1.8674x over previous
import functools

import jax
import jax.numpy as jnp
from jax import lax
from jax.experimental import pallas as pl
from jax.experimental.pallas import tpu as pltpu
from jax.experimental.pallas import tpu_sc as plsc

DIM = 64
MARGIN = 1.0
BATCH = 16384
CHUNK = 32
LANES = 16


def _rsqrt(x):
    i = plsc.bitcast(x, jnp.int32)
    y = plsc.bitcast(jnp.int32(0x5F3759DF) - (i >> 1), jnp.float32)
    for _ in range(3):
        y = y * (1.5 - 0.5 * x * y * y)
    return y


def _make_sc_kernel(num_workers, bpw):
    mesh = plsc.VectorSubcoreMesh(core_axis_name="c", subcore_axis_name="s")
    num_cores = mesh.num_cores
    nchunk = bpw // CHUNK

    @functools.partial(
        pl.kernel,
        mesh=mesh,
        compiler_params=pltpu.CompilerParams(needs_layout_passes=False),
        out_type=jax.ShapeDtypeStruct((num_workers, 128), jnp.float32),
        scratch_types=[
            pltpu.VMEM((bpw,), jnp.int32),
            pltpu.VMEM((bpw,), jnp.int32),
            pltpu.VMEM((bpw,), jnp.int32),
            pltpu.VMEM((CHUNK, 8, DIM), jnp.float32),
            pltpu.VMEM((CHUNK, 8, DIM), jnp.float32),
            pltpu.VMEM((CHUNK, 8, DIM), jnp.float32),
            pltpu.VMEM((128,), jnp.float32),
            pltpu.SemaphoreType.DMA,
        ],
    )
    def trans_e_cost(lidx_hbm, ridx_hbm, qidx_hbm, ent_hbm, rel_hbm,
                     out_hbm, lv, rv, qv,
                     ltiles, rtiles, qtiles, outv, sem):
        wid = lax.axis_index("s") * num_cores + lax.axis_index("c")
        base = wid * bpw
        pltpu.sync_copy(lidx_hbm.at[pl.ds(base, bpw)], lv)
        pltpu.sync_copy(ridx_hbm.at[pl.ds(base, bpw)], rv)
        pltpu.sync_copy(qidx_hbm.at[pl.ds(base, bpw)], qv)

        iota = lax.iota(jnp.int32, LANES)
        zero = jnp.zeros((LANES,), jnp.float32)

        def chunk_body(ch, acc):
            off = ch * CHUNK
            copies = []
            for k in range(CHUNK // LANES):
                lslab = lv[pl.ds(off + k * LANES, LANES)] >> 3
                rslab = rv[pl.ds(off + k * LANES, LANES)] >> 3
                qslab = qv[pl.ds(off + k * LANES, LANES)] >> 3
                for j in range(LANES):
                    kk = k * LANES + j
                    copies.append(pltpu.async_copy(
                        ent_hbm.at[lslab[j]], ltiles.at[kk], sem))
                    copies.append(pltpu.async_copy(
                        ent_hbm.at[rslab[j]], rtiles.at[kk], sem))
                    copies.append(pltpu.async_copy(
                        rel_hbm.at[qslab[j]], qtiles.at[kk], sem))
            for cpy in copies:
                cpy.wait()
            for g in range(CHUNK // LANES):
                rowloc = g * LANES + iota
                lsub = lv[pl.ds(off + g * LANES, LANES)] & 7
                rsub = rv[pl.ds(off + g * LANES, LANES)] & 7
                qsub = qv[pl.ds(off + g * LANES, LANES)] & 7
                sl = sr = sq = dlr = dqr = zero
                for c in range(DIM):
                    ci = jnp.full((LANES,), c, jnp.int32)
                    lc = plsc.load_gather(ltiles, [rowloc, lsub, ci])
                    rc = plsc.load_gather(rtiles, [rowloc, rsub, ci])
                    qc = plsc.load_gather(qtiles, [rowloc, qsub, ci])
                    sl = sl + lc * lc
                    sr = sr + rc * rc
                    sq = sq + qc * qc
                    dlr = dlr + lc * rc
                    dqr = dqr + qc * rc
                tiny = jnp.float32(1e-24)
                simi = (dlr * _rsqrt(jnp.maximum(sl * sr, tiny))
                        + dqr * _rsqrt(jnp.maximum(sq * sr, tiny)))
                similn = simi
                simirn = simi
                costl = jnp.maximum(similn - simi + MARGIN, 0.0)
                costr = jnp.maximum(simirn - simi + MARGIN, 0.0)
                acc = acc + costl + costr
            return acc

        acc = lax.fori_loop(0, nchunk, chunk_body, zero)
        total = jnp.sum(acc) * jnp.float32(1.0 / BATCH)
        outlane = jnp.where(iota == 0, total, 0.0)
        for k in range(128 // LANES):
            outv[pl.ds(k * LANES, LANES)] = outlane if k == 0 else zero
        pltpu.sync_copy(outv, out_hbm.at[wid])

    return trans_e_cost


def kernel(leftEnIndices, rightEnIndices, relIndices, negLeftEnIndices,
           negRightEnIndices, entityEmbedding, relationEmbedding):
    del negLeftEnIndices, negRightEnIndices
    info = plsc.get_sparse_core_info()
    num_workers = info.num_cores * info.num_subcores
    bpw = BATCH // num_workers
    nent, dim = entityEmbedding.shape
    nrel = relationEmbedding.shape[0]
    ent3 = jnp.reshape(entityEmbedding, (nent // 8, 8, dim))
    rel3 = jnp.reshape(relationEmbedding, (nrel // 8, 8, dim))
    sc = _make_sc_kernel(num_workers, bpw)
    partials = sc(leftEnIndices.astype(jnp.int32),
                  rightEnIndices.astype(jnp.int32),
                  relIndices.astype(jnp.int32), ent3, rel3)
    return jnp.sum(partials)
